# baseline (speedup 1.0000x reference)
import functools

import jax
import jax.numpy as jnp
from jax import lax
from jax.experimental import pallas as pl
from jax.experimental.pallas import tpu as pltpu
from jax.experimental.pallas import tpu_sc as plsc

B = 128
V = 100000
CHUNK = 12544
GRID = 8


def _extract_windows(outputs, targets):
    nops = 16
    rows_per_step = B // 8

    def body(t_ref, *refs):
        xs, w_ref = refs[:nops], refs[nops]
        for k in range(nops):
            w_ref[pl.ds(k, 1), :] = xs[k][pl.ds(k % 8, 1), :]

    def make_map(k):
        return lambda i, t: ((i * rows_per_step + k) // 8,
                             t[i * rows_per_step + k] // 128)

    grid_spec = pltpu.PrefetchScalarGridSpec(
        num_scalar_prefetch=1,
        grid=(8,),
        in_specs=[pl.BlockSpec((8, 128), make_map(k)) for k in range(nops)],
        out_specs=pl.BlockSpec((rows_per_step, 128), lambda i, t: (i, 0)),
    )
    return pl.pallas_call(
        body,
        grid_spec=grid_spec,
        out_shape=jax.ShapeDtypeStruct((B, 128), jnp.float32),
    )(targets, *([outputs] * nops))


def _gather_target_vals(w, targets):
    info = plsc.get_sparse_core_info()
    nc = info.num_cores
    table = w.reshape(B * 8, 16)
    mesh = plsc.VectorSubcoreMesh(core_axis_name="c", subcore_axis_name="s")

    @functools.partial(
        pl.kernel,
        mesh=mesh,
        out_type=jax.ShapeDtypeStruct((B,), jnp.float32),
        compiler_params=pltpu.CompilerParams(use_tc_tiling_on_sc=False),
        scratch_types=[
            pltpu.VMEM((16,), jnp.int32),
            pltpu.VMEM((16, 16), jnp.float32),
            pltpu.VMEM((16,), jnp.float32),
            pltpu.SemaphoreType.DMA,
        ],
    )
    def gk(table_hbm, tgt_hbm, v_hbm, tgt_v, rows_v, out_v, sem):
        wid = lax.axis_index("s") * nc + lax.axis_index("c")

        @pl.when(wid < B // 16)
        def _():
            base = wid * 16
            pltpu.sync_copy(tgt_hbm.at[pl.ds(base, 16)], tgt_v)
            c = lax.bitwise_and(tgt_v[...], 127)
            rows = (base + lax.iota(jnp.int32, 16)) * 8 + \
                lax.shift_right_logical(c, 4)
            cl = lax.bitwise_and(c, 15)
            pltpu.async_copy(table_hbm.at[rows], rows_v, sem).wait()
            lane = lax.iota(jnp.int32, 16)
            acc = jnp.zeros((16,), jnp.float32)
            for l in range(16):
                g = rows_v[l, :].at[cl].get(mode="promise_in_bounds")
                acc = jnp.where(lane == l, g, acc)
            out_v[...] = acc
            pltpu.sync_copy(out_v, v_hbm.at[pl.ds(base, 16)])

    return gk(table, targets)


RB = 8
RGRID = B // RB


def _count_body(x_ref, v_ref, t_ref, out1_ref, out5_ref, a1_ref, a5_ref):
    j = pl.program_id(0)

    @pl.when(j == 0)
    def _():
        a1_ref[0] = 0.0
        a5_ref[0] = 0.0

    x = x_ref[...]
    v = v_ref[...]
    t = t_ref[...]
    col = lax.broadcasted_iota(jnp.int32, (RB, V), 1)
    m = (x > v) | ((x == v) & (col < t))
    rank = jnp.sum(m, axis=1, keepdims=True)
    a1_ref[0] += jnp.sum((rank < 1).astype(jnp.float32))
    a5_ref[0] += jnp.sum((rank < 5).astype(jnp.float32))

    @pl.when(j == RGRID - 1)
    def _():
        out1_ref[0] = a1_ref[0] * (100.0 / B)
        out5_ref[0] = a5_ref[0] * (100.0 / B)


def kernel(outputs, targets):
    v = jnp.zeros((B,), jnp.float32)
    out1, out5 = pl.pallas_call(
        _count_body,
        grid=(RGRID,),
        in_specs=[
            pl.BlockSpec((RB, V), lambda j: (j, 0)),
            pl.BlockSpec((RB, 1), lambda j: (j, 0)),
            pl.BlockSpec((RB, 1), lambda j: (j, 0)),
        ],
        out_specs=[
            pl.BlockSpec(memory_space=pltpu.SMEM),
            pl.BlockSpec(memory_space=pltpu.SMEM),
        ],
        out_shape=[
            jax.ShapeDtypeStruct((1,), jnp.float32),
            jax.ShapeDtypeStruct((1,), jnp.float32),
        ],
        scratch_shapes=[pltpu.SMEM((1,), jnp.float32),
                        pltpu.SMEM((1,), jnp.float32)],
    )(outputs, v.reshape(B, 1), targets.reshape(B, 1))
    return (out1, out5)


# speedup vs baseline: 1.7518x; 1.7518x over previous
import functools

import jax
import jax.numpy as jnp
from jax import lax
from jax.experimental import pallas as pl
from jax.experimental.pallas import tpu as pltpu
from jax.experimental.pallas import tpu_sc as plsc

B = 128
V = 100000
CV = 10000
GRID = V // CV


def _gather_target_vals(xt, targets):
    info = plsc.get_sparse_core_info()
    nc = info.num_cores
    mesh = plsc.VectorSubcoreMesh(core_axis_name="c", subcore_axis_name="s")

    @functools.partial(
        pl.kernel,
        mesh=mesh,
        out_type=jax.ShapeDtypeStruct((B,), jnp.float32),
        compiler_params=pltpu.CompilerParams(use_tc_tiling_on_sc=False),
        scratch_types=[
            pltpu.VMEM((16,), jnp.int32),
            pltpu.VMEM((16, B), jnp.float32),
            pltpu.VMEM((16,), jnp.float32),
            pltpu.SemaphoreType.DMA,
        ],
    )
    def gk(xt_hbm, tgt_hbm, v_hbm, tgt_v, rows_v, out_v, sem):
        wid = lax.axis_index("s") * nc + lax.axis_index("c")

        @pl.when(wid < B // 16)
        def _():
            base = wid * 16
            pltpu.sync_copy(tgt_hbm.at[pl.ds(base, 16)], tgt_v)
            pltpu.async_copy(xt_hbm.at[tgt_v[...]], rows_v, sem).wait()
            lane = lax.iota(jnp.int32, 16)
            acc = jnp.zeros((16,), jnp.float32)
            for l in range(16):
                r = base + l
                part = rows_v[l, pl.ds((r // 16) * 16, 16)]
                idx = jnp.full((16,), r % 16, jnp.int32)
                g = part.at[idx].get(mode="promise_in_bounds")
                acc = jnp.where(lane == l, g, acc)
            out_v[...] = acc
            pltpu.sync_copy(out_v, v_hbm.at[pl.ds(base, 16)])

    return gk(xt, targets)


def _count_body(x_ref, v_ref, t_ref, out1_ref, out5_ref, acc_ref):
    j = pl.program_id(0)

    @pl.when(j == 0)
    def _():
        acc_ref[...] = jnp.zeros_like(acc_ref)

    x = x_ref[...]
    v = v_ref[...]
    t = t_ref[...]
    row = j * CV + lax.broadcasted_iota(jnp.int32, (CV, B), 0)
    m = (x > v) | ((x == v) & (row < t))
    acc_ref[...] += jnp.sum(m, axis=0, keepdims=True)

    @pl.when(j == GRID - 1)
    def _():
        rank = acc_ref[...]
        out1_ref[0] = jnp.sum((rank < 1).astype(jnp.float32)) * (100.0 / B)
        out5_ref[0] = jnp.sum((rank < 5).astype(jnp.float32)) * (100.0 / B)


def kernel(outputs, targets):
    xt = outputs.T
    v = _gather_target_vals(xt, targets)
    out1, out5 = pl.pallas_call(
        _count_body,
        grid=(GRID,),
        in_specs=[
            pl.BlockSpec((CV, B), lambda j: (j, 0)),
            pl.BlockSpec((1, B), lambda j: (0, 0)),
            pl.BlockSpec((1, B), lambda j: (0, 0)),
        ],
        out_specs=[
            pl.BlockSpec(memory_space=pltpu.SMEM),
            pl.BlockSpec(memory_space=pltpu.SMEM),
        ],
        out_shape=[
            jax.ShapeDtypeStruct((1,), jnp.float32),
            jax.ShapeDtypeStruct((1,), jnp.float32),
        ],
        scratch_shapes=[pltpu.VMEM((1, B), jnp.int32)],
    )(xt, v.reshape(1, B), targets.reshape(1, B))
    return (out1, out5)
